# baseline (speedup 1.0000x reference)
import functools

import jax
import jax.numpy as jnp
import numpy as np
from jax import lax
from jax.experimental import pallas as pl
from jax.experimental.pallas import tpu as pltpu
from jax.experimental.pallas import tpu_sc as plsc

_B, _HOUT, _WOUT = 4, 512, 512
_NS = _B * _HOUT * _WOUT
_NW = 32
_PW = _NS // _NW
_S = 2048
_NCHUNK = _PW // _S
_LEVELS = (4096, 2048, 1024, 512)
_HI = np.int32(-65536)


def _tec_body(p1, p2, p3, p4, uvt_hbm, out_hbm,
              u_ab, v_ab, uvi_ab, vvi_ab,
              idx_a, val_a, w_a, idx_b, val_b, w_b,
              o_ab, sem_g, sem_uv, sem_o):
    ptabs = (p1, p2, p3, p4)
    bufs = ((idx_a, val_a, w_a, sem_g[0]), (idx_b, val_b, w_b, sem_g[1]))
    wid = lax.axis_index("s") * 2 + lax.axis_index("c")
    wbase = wid * _PW

    lane2 = lax.iota(jnp.int32, 16) * 2

    def uv_copies(p):
        return (pltpu.make_async_copy(
                    uvt_hbm.at[uvi_ab[p]], u_ab[p], sem_uv[p]),
                pltpu.make_async_copy(
                    uvt_hbm.at[vvi_ab[p]], v_ab[p], sem_uv[p]))

    def fire_uv(c, p):
        c0 = (wbase + c * _S) * 2

        @pl.loop(0, _S // 16)
        def _(j):
            iu = lane2 + (c0 + j * 32)
            uvi_ab[p][pl.ds(j * 16, 16)] = iu
            vvi_ab[p][pl.ds(j * 16, 16)] = iu + 1

        for cp in uv_copies(p):
            cp.start()

    def drain_uv(c, p):
        for cp in uv_copies(p):
            cp.wait()

    def compute(p):
        idx_refs, _, w_v, _ = bufs[p]
        u_v, v_v = u_ab[p], v_ab[p]

        @pl.loop(0, _S // 16)
        def _(j):
            off = j * 16
            u = u_v[pl.ds(off, 16)]
            v = v_v[pl.ds(off, 16)]
            gx = u * 2.0 - 1.0
            gy = v * 2.0 - 1.0
            for l, n in enumerate(_LEVELS):
                nf = float(n)
                ix = ((gx + 1.0) * nf - 1.0) / 2.0
                iy = ((gy + 1.0) * nf - 1.0) / 2.0
                x0p1 = (ix + 1.0).astype(jnp.int32)
                y0p1 = (iy + 1.0).astype(jnp.int32)
                ix0 = x0p1 - 1
                iy0 = y0p1 - 1
                wx1 = ix - ix0.astype(jnp.float32)
                wy1 = iy - iy0.astype(jnp.float32)
                wx0 = 1.0 - wx1
                wy0 = 1.0 - wy1
                wx0 = jnp.where(ix0 >= 0, wx0, 0.0)
                wx1 = jnp.where(x0p1 <= n - 1, wx1, 0.0)
                wy0 = jnp.where(iy0 >= 0, wy0, 0.0)
                wy1 = jnp.where(y0p1 <= n - 1, wy1, 0.0)
                yc0 = jnp.maximum(iy0, 0)
                yc1 = jnp.minimum(y0p1, n - 1)
                q0 = yc0 * n + x0p1
                q1 = yc1 * n + x0p1
                idx_refs[2 * l + 0][pl.ds(off, 16)] = q0
                idx_refs[2 * l + 1][pl.ds(off, 16)] = q1

                def wpack(walo, wahi):
                    pa = lax.bitcast_convert_type(walo, jnp.int32)
                    pb = lax.bitcast_convert_type(wahi, jnp.int32)
                    return lax.bitwise_or(
                        lax.bitwise_and(
                            lax.shift_right_logical(pa, 16), jnp.int32(65535)),
                        lax.bitwise_and(pb, _HI))

                w_v[2 * l + 0, pl.ds(off, 16)] = wpack(wy0 * wx0, wy0 * wx1)
                w_v[2 * l + 1, pl.ds(off, 16)] = wpack(wy1 * wx0, wy1 * wx1)

    def g_copies(p):
        idx_refs, val_refs, _, sem = bufs[p]
        return [
            pltpu.make_async_copy(
                ptabs[r // 2].at[idx_refs[r]], val_refs[r], sem)
            for r in range(8)
        ]

    def fire(p):
        for cp in g_copies(p):
            cp.start()

    def drain(p):
        for cp in g_copies(p):
            cp.wait()

    def o_copy(c, p):
        return pltpu.make_async_copy(
            o_ab[p], out_hbm.at[pl.ds(wbase + c * _S, _S)], sem_o[p])

    def combine_store(c, p):
        _, val_refs, w_v, _ = bufs[p]
        o_v = o_ab[p]

        @pl.when(c >= 2)
        def _():
            o_copy(c - 2, p).wait()

        @pl.loop(0, _S // 16)
        def _(j):
            off = j * 16
            acc = None
            for l in range(4):
                xi0 = val_refs[2 * l + 0][pl.ds(off, 16)]
                xi1 = val_refs[2 * l + 1][pl.ds(off, 16)]
                wp0 = w_v[2 * l + 0, pl.ds(off, 16)]
                wp1 = w_v[2 * l + 1, pl.ds(off, 16)]
                v00 = lax.bitcast_convert_type(lax.shift_left(xi0, 16), jnp.float32)
                v01 = lax.bitcast_convert_type(lax.bitwise_and(xi0, _HI), jnp.float32)
                v10 = lax.bitcast_convert_type(lax.shift_left(xi1, 16), jnp.float32)
                v11 = lax.bitcast_convert_type(lax.bitwise_and(xi1, _HI), jnp.float32)
                w00 = lax.bitcast_convert_type(lax.shift_left(wp0, 16), jnp.float32)
                w01 = lax.bitcast_convert_type(lax.bitwise_and(wp0, _HI), jnp.float32)
                w10 = lax.bitcast_convert_type(lax.shift_left(wp1, 16), jnp.float32)
                w11 = lax.bitcast_convert_type(lax.bitwise_and(wp1, _HI), jnp.float32)
                s = v00 * w00 + v01 * w01 + v10 * w10 + v11 * w11
                acc = s if acc is None else acc + s
            o_v[pl.ds(off, 16)] = acc

        o_copy(c, p).start()

    fire_uv(0, 0)
    fire_uv(1, 1)
    drain_uv(0, 0)
    compute(0)
    fire(0)

    @pl.loop(0, _NCHUNK // 2)
    def _(tt):
        i = tt * 2 + 1
        drain_uv(i, 1)
        compute(1)
        fire(1)

        @pl.when(i + 1 < _NCHUNK)
        def _():
            fire_uv(i + 1, 0)
        drain(0)
        combine_store(i - 1, 0)

        @pl.when(tt < _NCHUNK // 2 - 1)
        def _():
            i2 = i + 1
            drain_uv(i2, 0)
            compute(0)
            fire(0)
            fire_uv(i2 + 1, 1)
            drain(1)
            combine_store(i2 - 1, 1)

    drain(1)
    combine_store(_NCHUNK - 1, 1)
    o_copy(_NCHUNK - 2, 0).wait()
    o_copy(_NCHUNK - 1, 1).wait()


@functools.partial(
    pl.kernel,
    out_type=jax.ShapeDtypeStruct((_NS,), jnp.float32),
    mesh=plsc.VectorSubcoreMesh(core_axis_name="c", subcore_axis_name="s"),
    scratch_types=[
        [pltpu.VMEM((_S,), jnp.float32)] * 2,
        [pltpu.VMEM((_S,), jnp.float32)] * 2,
        [pltpu.VMEM((_S,), jnp.int32)] * 2,
        [pltpu.VMEM((_S,), jnp.int32)] * 2,
        [pltpu.VMEM((_S,), jnp.int32)] * 8,
        [pltpu.VMEM((_S,), jnp.int32)] * 8,
        pltpu.VMEM((8, _S), jnp.int32),
        [pltpu.VMEM((_S,), jnp.int32)] * 8,
        [pltpu.VMEM((_S,), jnp.int32)] * 8,
        pltpu.VMEM((8, _S), jnp.int32),
        [pltpu.VMEM((_S,), jnp.float32)] * 2,
        [pltpu.SemaphoreType.DMA] * 2,
        [pltpu.SemaphoreType.DMA] * 2,
        [pltpu.SemaphoreType.DMA] * 2,
    ],
)
def _sc_sample(p1, p2, p3, p4, uvt_hbm, out_hbm,
               u_ab, v_ab, uvi_ab, vvi_ab,
               idx_a, val_a, w_a, idx_b, val_b, w_b,
               o_ab, sem_g, sem_uv, sem_o):
    _tec_body(p1, p2, p3, p4, uvt_hbm, out_hbm,
              u_ab, v_ab, uvi_ab, vvi_ab,
              idx_a, val_a, w_a, idx_b, val_b, w_b,
              o_ab, sem_g, sem_uv, sem_o)


def _pair_table(layer):
    flat = layer.reshape(-1)
    bits = lax.bitcast_convert_type(
        flat.astype(jnp.bfloat16), jnp.uint16).astype(jnp.uint32)
    z = jnp.zeros((1,), jnp.uint32)
    lo = jnp.concatenate([z, bits])
    hi = jnp.concatenate([bits, z])
    return lax.bitcast_convert_type(lo | (hi << 16), jnp.int32)


@jax.jit
def kernel(uv, layer1, layer2, layer3, layer4):
    out = _sc_sample(
        _pair_table(layer1), _pair_table(layer2),
        _pair_table(layer3), _pair_table(layer4),
        uv.reshape(-1),
    )
    return out.reshape(_B, 1, _HOUT, _WOUT)


# speedup vs baseline: 3.6152x; 3.6152x over previous
import functools

import jax
import jax.numpy as jnp
import numpy as np
from jax import lax
from jax.experimental import pallas as pl
from jax.experimental.pallas import tpu as pltpu
from jax.experimental.pallas import tpu_sc as plsc

_B, _HOUT, _WOUT = 4, 512, 512
_NS = _B * _HOUT * _WOUT
_NW = 32
_PW = _NS // _NW
_S = 2048
_NCHUNK = _PW // _S
_LEVELS = (4096, 2048, 1024, 512)
_HI = np.int32(-65536)


def _tec_body(p1, p2, p3, p4, uvt_hbm, out_hbm,
              u_ab, v_ab,
              idx_a, val_a, w_a, idx_b, val_b, w_b,
              o_ab, sem_g, sem_uv, sem_o):
    ptabs = (p1, p2, p3, p4)
    bufs = ((idx_a, val_a, w_a, sem_g[0]), (idx_b, val_b, w_b, sem_g[1]))
    wid = lax.axis_index("s") * 2 + lax.axis_index("c")
    wbase = wid * _PW

    def uv_copies(c, p):
        base = wbase + c * _S
        return (pltpu.make_async_copy(
                    uvt_hbm.at[0, pl.ds(base, _S)], u_ab[p], sem_uv[p]),
                pltpu.make_async_copy(
                    uvt_hbm.at[1, pl.ds(base, _S)], v_ab[p], sem_uv[p]))

    def fire_uv(c, p):
        for cp in uv_copies(c, p):
            cp.start()

    def drain_uv(c, p):
        for cp in uv_copies(c, p):
            cp.wait()

    def compute(p):
        idx_refs, _, w_v, _ = bufs[p]
        u_v, v_v = u_ab[p], v_ab[p]

        @pl.loop(0, _S // 16)
        def _(j):
            off = j * 16
            u = u_v[pl.ds(off, 16)]
            v = v_v[pl.ds(off, 16)]
            gx = u * 2.0 - 1.0
            gy = v * 2.0 - 1.0
            for l, n in enumerate(_LEVELS):
                nf = float(n)
                ix = ((gx + 1.0) * nf - 1.0) / 2.0
                iy = ((gy + 1.0) * nf - 1.0) / 2.0
                x0p1 = (ix + 1.0).astype(jnp.int32)
                y0p1 = (iy + 1.0).astype(jnp.int32)
                ix0 = x0p1 - 1
                iy0 = y0p1 - 1
                wx1 = ix - ix0.astype(jnp.float32)
                wy1 = iy - iy0.astype(jnp.float32)
                wx0 = 1.0 - wx1
                wy0 = 1.0 - wy1
                wx0 = jnp.where(ix0 >= 0, wx0, 0.0)
                wx1 = jnp.where(x0p1 <= n - 1, wx1, 0.0)
                wy0 = jnp.where(iy0 >= 0, wy0, 0.0)
                wy1 = jnp.where(y0p1 <= n - 1, wy1, 0.0)
                yc0 = jnp.maximum(iy0, 0)
                yc1 = jnp.minimum(y0p1, n - 1)
                q0 = yc0 * n + x0p1
                q1 = yc1 * n + x0p1
                idx_refs[2 * l + 0][pl.ds(off, 16)] = q0
                idx_refs[2 * l + 1][pl.ds(off, 16)] = q1

                def wpack(walo, wahi):
                    pa = lax.bitcast_convert_type(walo, jnp.int32)
                    pb = lax.bitcast_convert_type(wahi, jnp.int32)
                    return lax.bitwise_or(
                        lax.bitwise_and(
                            lax.shift_right_logical(pa, 16), jnp.int32(65535)),
                        lax.bitwise_and(pb, _HI))

                w_v[2 * l + 0, pl.ds(off, 16)] = wpack(wy0 * wx0, wy0 * wx1)
                w_v[2 * l + 1, pl.ds(off, 16)] = wpack(wy1 * wx0, wy1 * wx1)

    def g_copies(p):
        idx_refs, val_refs, _, sem = bufs[p]
        return [
            pltpu.make_async_copy(
                ptabs[r // 2].at[idx_refs[r]], val_refs[r], sem)
            for r in range(8)
        ]

    def fire(p):
        for cp in g_copies(p):
            cp.start()

    def drain(p):
        for cp in g_copies(p):
            cp.wait()

    def o_copy(c, p):
        return pltpu.make_async_copy(
            o_ab[p], out_hbm.at[pl.ds(wbase + c * _S, _S)], sem_o[p])

    def combine_store(c, p):
        _, val_refs, w_v, _ = bufs[p]
        o_v = o_ab[p]

        @pl.when(c >= 2)
        def _():
            o_copy(c - 2, p).wait()

        @pl.loop(0, _S // 16)
        def _(j):
            off = j * 16
            acc = None
            for l in range(4):
                xi0 = val_refs[2 * l + 0][pl.ds(off, 16)]
                xi1 = val_refs[2 * l + 1][pl.ds(off, 16)]
                wp0 = w_v[2 * l + 0, pl.ds(off, 16)]
                wp1 = w_v[2 * l + 1, pl.ds(off, 16)]
                v00 = lax.bitcast_convert_type(lax.shift_left(xi0, 16), jnp.float32)
                v01 = lax.bitcast_convert_type(lax.bitwise_and(xi0, _HI), jnp.float32)
                v10 = lax.bitcast_convert_type(lax.shift_left(xi1, 16), jnp.float32)
                v11 = lax.bitcast_convert_type(lax.bitwise_and(xi1, _HI), jnp.float32)
                w00 = lax.bitcast_convert_type(lax.shift_left(wp0, 16), jnp.float32)
                w01 = lax.bitcast_convert_type(lax.bitwise_and(wp0, _HI), jnp.float32)
                w10 = lax.bitcast_convert_type(lax.shift_left(wp1, 16), jnp.float32)
                w11 = lax.bitcast_convert_type(lax.bitwise_and(wp1, _HI), jnp.float32)
                s = v00 * w00 + v01 * w01 + v10 * w10 + v11 * w11
                acc = s if acc is None else acc + s
            o_v[pl.ds(off, 16)] = acc

        o_copy(c, p).start()

    fire_uv(0, 0)
    fire_uv(1, 1)
    drain_uv(0, 0)
    compute(0)
    fire(0)

    @pl.loop(0, _NCHUNK // 2)
    def _(tt):
        i = tt * 2 + 1
        drain_uv(i, 1)
        compute(1)
        fire(1)

        @pl.when(i + 1 < _NCHUNK)
        def _():
            fire_uv(i + 1, 0)
        drain(0)
        combine_store(i - 1, 0)

        @pl.when(tt < _NCHUNK // 2 - 1)
        def _():
            i2 = i + 1
            drain_uv(i2, 0)
            compute(0)
            fire(0)
            fire_uv(i2 + 1, 1)
            drain(1)
            combine_store(i2 - 1, 1)

    drain(1)
    combine_store(_NCHUNK - 1, 1)
    o_copy(_NCHUNK - 2, 0).wait()
    o_copy(_NCHUNK - 1, 1).wait()


@functools.partial(
    pl.kernel,
    out_type=jax.ShapeDtypeStruct((_NS,), jnp.float32),
    mesh=plsc.VectorSubcoreMesh(core_axis_name="c", subcore_axis_name="s"),
    scratch_types=[
        [pltpu.VMEM((_S,), jnp.float32)] * 2,
        [pltpu.VMEM((_S,), jnp.float32)] * 2,
        [pltpu.VMEM((_S,), jnp.int32)] * 8,
        [pltpu.VMEM((_S,), jnp.int32)] * 8,
        pltpu.VMEM((8, _S), jnp.int32),
        [pltpu.VMEM((_S,), jnp.int32)] * 8,
        [pltpu.VMEM((_S,), jnp.int32)] * 8,
        pltpu.VMEM((8, _S), jnp.int32),
        [pltpu.VMEM((_S,), jnp.float32)] * 2,
        [pltpu.SemaphoreType.DMA] * 2,
        [pltpu.SemaphoreType.DMA] * 2,
        [pltpu.SemaphoreType.DMA] * 2,
    ],
)
def _sc_sample(p1, p2, p3, p4, uvt_hbm, out_hbm,
               u_ab, v_ab,
               idx_a, val_a, w_a, idx_b, val_b, w_b,
               o_ab, sem_g, sem_uv, sem_o):
    _tec_body(p1, p2, p3, p4, uvt_hbm, out_hbm,
              u_ab, v_ab,
              idx_a, val_a, w_a, idx_b, val_b, w_b,
              o_ab, sem_g, sem_uv, sem_o)


def _pair_table(layer):
    flat = layer.reshape(-1)
    bits = lax.bitcast_convert_type(
        flat.astype(jnp.bfloat16), jnp.uint16).astype(jnp.uint32)
    z = jnp.zeros((1,), jnp.uint32)
    lo = jnp.concatenate([z, bits])
    hi = jnp.concatenate([bits, z])
    return lax.bitcast_convert_type(lo | (hi << 16), jnp.int32)


@jax.jit
def kernel(uv, layer1, layer2, layer3, layer4):
    uvt = uv.reshape(_NS, 2).T
    out = _sc_sample(
        _pair_table(layer1), _pair_table(layer2),
        _pair_table(layer3), _pair_table(layer4),
        uvt,
    )
    return out.reshape(_B, 1, _HOUT, _WOUT)
